# Initial kernel scaffold; baseline (speedup 1.0000x reference)
import jax
import jax.numpy as jnp
from jax.experimental import pallas as pl


def kernel(x, table):
    raise NotImplementedError("write your pallas kernel here")


# speedup vs baseline: 2.4230x; 2.4230x over previous
import functools
import math

import jax
import jax.numpy as jnp
from jax import lax
from jax.experimental import pallas as pl
from jax.experimental.pallas import tpu as pltpu
from jax.experimental.pallas import tpu_sc as plsc

D = 128
L = 16
NC, NS = 2, 16
NW = NC * NS
CHUNK = 128
SCALE = math.sqrt(D)


def _emb_body(idx_hbm, table_hbm, out_hbm, idx_v, rows_v, gsem):
    wid = lax.axis_index("s") * NC + lax.axis_index("c")
    n_chunks = idx_v.shape[0]
    base = wid * (n_chunks * CHUNK)

    pltpu.sync_copy(idx_hbm.at[wid], idx_v)

    def chunk(j, carry):
        pltpu.async_copy(table_hbm.at[idx_v.at[j]], rows_v, gsem).wait()

        @plsc.parallel_loop(0, CHUNK)
        def scale_row(r):
            for c in range(D // L):
                rows_v[r, pl.ds(c * L, L)] = rows_v[r, pl.ds(c * L, L)] * SCALE

        pltpu.sync_copy(rows_v, out_hbm.at[pl.ds(base + j * CHUNK, CHUNK)])
        return carry

    lax.fori_loop(0, n_chunks, chunk, 0)


def kernel(x, table):
    B, S = x.shape
    n_tok = B * S
    n_per = n_tok // NW
    n_chunks = n_per // CHUNK
    idx3d = x.reshape(NW, n_chunks, CHUNK).astype(jnp.int32)

    run = functools.partial(
        pl.kernel,
        out_type=jax.ShapeDtypeStruct((n_tok, D), jnp.float32),
        mesh=plsc.VectorSubcoreMesh(
            core_axis_name="c", subcore_axis_name="s",
            num_cores=NC, num_subcores=NS,
        ),
        scratch_types=[
            pltpu.VMEM((n_chunks, CHUNK), jnp.int32),
            pltpu.VMEM((CHUNK, D), jnp.float32),
            pltpu.SemaphoreType.DMA,
        ],
    )(_emb_body)

    out = run(idx3d, table)
    return out.reshape(B, S, D)


# speedup vs baseline: 2.9659x; 1.2241x over previous
import functools
import math

import jax
import jax.numpy as jnp
from jax import lax
from jax.experimental import pallas as pl
from jax.experimental.pallas import tpu as pltpu
from jax.experimental.pallas import tpu_sc as plsc

D = 128
L = 16
NC, NS = 2, 16
NW = NC * NS
CHUNK = 128
NBUF = 5
LEAD = 3
SCALE = math.sqrt(D)


def _emb_body(idx_hbm, table_hbm, out_hbm, idx_v, rows_v, gsem, ssem):
    wid = lax.axis_index("s") * NC + lax.axis_index("c")
    n_chunks = idx_v.shape[0]
    base = wid * (n_chunks * CHUNK)

    pltpu.sync_copy(idx_hbm.at[wid], idx_v)

    def start_gather(j, b):
        pltpu.async_copy(table_hbm.at[idx_v.at[j]], rows_v.at[b], gsem.at[b])

    def wait_gather(b):
        pltpu.make_async_copy(
            table_hbm.at[idx_v.at[0]], rows_v.at[b], gsem.at[b]
        ).wait()

    def start_scatter(j, b):
        pltpu.async_copy(
            rows_v.at[b], out_hbm.at[pl.ds(base + j * CHUNK, CHUNK)], ssem.at[b]
        )

    def wait_scatter(b):
        pltpu.make_async_copy(
            rows_v.at[b], out_hbm.at[pl.ds(base, CHUNK)], ssem.at[b]
        ).wait()

    for b in range(LEAD):
        start_gather(b, b)

    def group(g, carry):
        for b in range(NBUF):
            j = g * NBUF + b
            jn = j + LEAD
            bn = (b + LEAD) % NBUF

            @pl.when(jn < n_chunks)
            def _():
                @pl.when(jn >= NBUF)
                def _():
                    wait_scatter(bn)
                start_gather(jn, bn)

            wait_gather(b)

            @plsc.parallel_loop(0, CHUNK)
            def scale_row(r):
                for c in range(D // L):
                    rows_v[b, r, pl.ds(c * L, L)] = (
                        rows_v[b, r, pl.ds(c * L, L)] * SCALE
                    )

            start_scatter(j, b)
        return carry

    lax.fori_loop(0, n_chunks // NBUF, group, 0)

    for b in range(NBUF):
        wait_scatter(b)


def kernel(x, table):
    B, S = x.shape
    n_tok = B * S
    n_per = n_tok // NW
    n_chunks = n_per // CHUNK
    idx3d = x.reshape(NW, n_chunks, CHUNK).astype(jnp.int32)

    run = functools.partial(
        pl.kernel,
        out_type=jax.ShapeDtypeStruct((n_tok, D), jnp.float32),
        mesh=plsc.VectorSubcoreMesh(
            core_axis_name="c", subcore_axis_name="s",
            num_cores=NC, num_subcores=NS,
        ),
        scratch_types=[
            pltpu.VMEM((n_chunks, CHUNK), jnp.int32),
            pltpu.VMEM((NBUF, CHUNK, D), jnp.float32),
            pltpu.SemaphoreType.DMA((NBUF,)),
            pltpu.SemaphoreType.DMA((NBUF,)),
        ],
    )(_emb_body)

    out = run(idx3d, table)
    return out.reshape(B, S, D)


# speedup vs baseline: 5.2384x; 1.7662x over previous
import functools
import math

import jax
import jax.numpy as jnp
from jax import lax
from jax.experimental import pallas as pl
from jax.experimental.pallas import tpu as pltpu
from jax.experimental.pallas import tpu_sc as plsc

D = 128
L = 16
NC, NS = 2, 16
NW = NC * NS
NB = 4
NBUF = 4
LEAD = 3
SCALE = math.sqrt(D)


def _emb_body(x_hbm, table_hbm, out_hbm, idx_v, rows_v, gsem, ssem):
    S = x_hbm.shape[1]
    b_per_w = x_hbm.shape[0] // NW
    n_chunks = b_per_w // NB
    wid = lax.axis_index("s") * NC + lax.axis_index("c")
    base = wid * b_per_w

    pltpu.sync_copy(x_hbm.at[pl.ds(base, b_per_w)], idx_v)

    def start_gather(j, b):
        for k in range(NB):
            pltpu.async_copy(
                table_hbm.at[idx_v.at[j * NB + k]], rows_v.at[b, k], gsem.at[b]
            )

    def wait_gather(b):
        for k in range(NB):
            pltpu.make_async_copy(
                table_hbm.at[idx_v.at[0]], rows_v.at[b, k], gsem.at[b]
            ).wait()

    def start_scatter(j, b):
        pltpu.async_copy(
            rows_v.at[b], out_hbm.at[pl.ds(base + j * NB, NB)], ssem.at[b]
        )

    def wait_scatter(b):
        pltpu.make_async_copy(
            rows_v.at[b], out_hbm.at[pl.ds(base, NB)], ssem.at[b]
        ).wait()

    for b in range(LEAD):
        start_gather(b, b)

    def group(g, carry):
        for b in range(NBUF):
            j = g * NBUF + b
            jn = j + LEAD
            bn = (b + LEAD) % NBUF

            @pl.when(jn < n_chunks)
            def _():
                @pl.when(jn >= NBUF)
                def _():
                    wait_scatter(bn)
                start_gather(jn, bn)

            wait_gather(b)

            @plsc.parallel_loop(0, S)
            def scale_row(r):
                for k in range(NB):
                    for c in range(D // L):
                        rows_v[b, k, r, pl.ds(c * L, L)] = (
                            rows_v[b, k, r, pl.ds(c * L, L)] * SCALE
                        )

            start_scatter(j, b)
        return carry

    lax.fori_loop(0, n_chunks // NBUF, group, 0)

    for b in range(NBUF):
        wait_scatter(b)


def kernel(x, table):
    B, S = x.shape
    V, d = table.shape

    run = functools.partial(
        pl.kernel,
        out_type=jax.ShapeDtypeStruct((B, S, d), jnp.float32),
        mesh=plsc.VectorSubcoreMesh(
            core_axis_name="c", subcore_axis_name="s",
            num_cores=NC, num_subcores=NS,
        ),
        scratch_types=[
            pltpu.VMEM((B // NW, S), jnp.int32),
            pltpu.VMEM((NBUF, NB, S, d), jnp.float32),
            pltpu.SemaphoreType.DMA((NBUF,)),
            pltpu.SemaphoreType.DMA((NBUF,)),
        ],
    )(_emb_body)

    return run(x.astype(jnp.int32), table)


# speedup vs baseline: 9.2246x; 1.7610x over previous
import functools
import math

import jax
import jax.numpy as jnp
from jax import lax
from jax.experimental import pallas as pl
from jax.experimental.pallas import tpu as pltpu
from jax.experimental.pallas import tpu_sc as plsc

D = 128
L = 16
NC, NS = 2, 16
NW = NC * NS
CHUNK = 128
NBUF = 5
LEAD = 3
SCALE = math.sqrt(D)


def _emb_body(idx_hbm, table_hbm, out_hbm, idx_v, rows_v, gsem, ssem):
    wid = lax.axis_index("s") * NC + lax.axis_index("c")
    n_chunks = idx_v.shape[0]
    base = wid * (n_chunks * CHUNK)

    pltpu.sync_copy(idx_hbm.at[wid], idx_v)

    def start_gather(j, b):
        pltpu.async_copy(table_hbm.at[idx_v.at[j]], rows_v.at[b], gsem.at[b])

    def wait_gather(b):
        pltpu.make_async_copy(
            table_hbm.at[idx_v.at[0]], rows_v.at[b], gsem.at[b]
        ).wait()

    def start_scatter(j, b):
        pltpu.async_copy(
            rows_v.at[b], out_hbm.at[pl.ds(base + j * CHUNK, CHUNK)], ssem.at[b]
        )

    def wait_scatter(b):
        pltpu.make_async_copy(
            rows_v.at[b], out_hbm.at[pl.ds(base, CHUNK)], ssem.at[b]
        ).wait()

    for b in range(LEAD):
        start_gather(b, b)

    def group(g, carry):
        for b in range(NBUF):
            j = g * NBUF + b
            jn = j + LEAD
            bn = (b + LEAD) % NBUF

            @pl.when(jn < n_chunks)
            def _():
                @pl.when(jn >= NBUF)
                def _():
                    wait_scatter(bn)
                start_gather(jn, bn)

            wait_gather(b)

            @plsc.parallel_loop(0, CHUNK)
            def scale_row(r):
                for c in range(D // L):
                    rows_v[b, r, pl.ds(c * L, L)] = (
                        rows_v[b, r, pl.ds(c * L, L)] * SCALE
                    )

            start_scatter(j, b)
        return carry

    lax.fori_loop(0, n_chunks // NBUF, group, 0)

    for b in range(NBUF):
        wait_scatter(b)


def kernel(x, table):
    B, S = x.shape
    n_tok = B * S
    n_per = n_tok // NW
    n_chunks = n_per // CHUNK
    idx3d = x.T.reshape(NW, n_chunks, CHUNK).astype(jnp.int32)

    run = functools.partial(
        pl.kernel,
        out_type=jax.ShapeDtypeStruct((n_tok, D), jnp.float32),
        mesh=plsc.VectorSubcoreMesh(
            core_axis_name="c", subcore_axis_name="s",
            num_cores=NC, num_subcores=NS,
        ),
        scratch_types=[
            pltpu.VMEM((n_chunks, CHUNK), jnp.int32),
            pltpu.VMEM((NBUF, CHUNK, D), jnp.float32),
            pltpu.SemaphoreType.DMA((NBUF,)),
            pltpu.SemaphoreType.DMA((NBUF,)),
        ],
    )(_emb_body)

    out = run(idx3d, table)
    return out.reshape(S, B, D).transpose(1, 0, 2)
